# baseline (speedup 1.0000x reference)
import functools

import jax
import jax.numpy as jnp
from jax import lax
from jax.experimental import pallas as pl
from jax.experimental.pallas import tpu as pltpu
from jax.experimental.pallas import tpu_sc as plsc

N = 10000
E = 640000
NUM_GRAPHS = 256
R = 5

NC = 2
NS = 16
NW = NC * NS
EPT = E // NW
EB = 160
NCHUNK = EPT // EB
NPAD = 10240
NPW = NPAD // NW
SEGPAD = 272

_F32 = jnp.float32

_SC_PARAMS = pltpu.CompilerParams(use_tc_tiling_on_sc=False,
                                  needs_layout_passes=False)


def _mesh():
    return plsc.VectorSubcoreMesh(
        core_axis_name="c", subcore_axis_name="s",
        num_cores=NC, num_subcores=NS)


def _gather0_body(t_y, t_r, xpad, ypad, rpad, idxv, bufy, bufr, sem):
    c = lax.axis_index("c")
    s = lax.axis_index("s")
    wid = s * NC + c
    base = wid * NPW
    pltpu.sync_copy(xpad.at[pl.ds(base, NPW)], idxv)
    pltpu.async_copy(t_y.at[idxv], bufy, sem).wait()
    pltpu.sync_copy(bufy, ypad.at[pl.ds(base, NPW)])
    pltpu.async_copy(t_r.at[idxv], bufr, sem).wait()
    pltpu.sync_copy(bufr, rpad.at[pl.ds(base, NPW)])


def _make_gather0(dyw, dr):
    return pl.kernel(
        _gather0_body,
        out_type=(jax.ShapeDtypeStruct((NPAD, dyw), jnp.int32),
                  jax.ShapeDtypeStruct((NPAD, dr), _F32)),
        mesh=_mesh(),
        compiler_params=_SC_PARAMS,
        scratch_types=[
            pltpu.VMEM((NPW,), jnp.int32),
            pltpu.VMEM((NPW, dyw), jnp.int32),
            pltpu.VMEM((NPW, dr), _F32),
            pltpu.SemaphoreType.DMA,
        ],
    )


GB = 80
SUPER = 4000
NSUP = EPT // SUPER
CPS = SUPER // GB
PAIRS = CPS // 2


def _edge_body(dout, y, src, dst, attr, zer, out,
               srcsup, dstsup, attrsup,
               rowsv0, rowsv1, msgv0, msgv1,
               dstr0, dstr1, agg,
               semg0, semg1, sems0, sems1):
    c = lax.axis_index("c")
    s = lax.axis_index("s")
    wid = s * NC + c
    rowsv = (rowsv0, rowsv1)
    msgv = (msgv0, msgv1)
    dstr = (dstr0, dstr1)
    semg = (semg0, semg1)
    sems = (sems0, sems1)

    @pl.when(s == 0)
    def _():
        pltpu.sync_copy(zer, agg)

    plsc.subcore_barrier()

    base0 = wid * EPT

    def sup_body(sup, carry):
        sbase = base0 + sup * SUPER
        pltpu.sync_copy(src.at[pl.ds(sbase, SUPER)], srcsup)
        pltpu.sync_copy(dst.at[pl.ds(sbase, SUPER)], dstsup)
        pltpu.sync_copy(attr.at[pl.ds(sbase * R, SUPER * R)],
                        attrsup.at[pl.ds(0, SUPER * R)])
        for b in range(2):
            pltpu.async_copy(y.at[srcsup.at[pl.ds(b * GB, GB)]],
                             rowsv[b], semg[b])

        def pair(p, carry2):
            for b in range(2):
                ch = 2 * p + b
                not_first = jnp.logical_or(sup > 0, ch >= 2)

                @pl.when(not_first)
                def _():
                    pltpu.make_async_copy(
                        msgv[b], agg.at[dstr[0]], sems[b]).wait()

                pltpu.make_async_copy(
                    y.at[srcsup.at[pl.ds(0, GB)]], rowsv[b], semg[b]).wait()

                co = ch * GB * R
                rw = dout // 2

                def edge(i, carry3):
                    av = attrsup[pl.ds(co + i * R, 16)]
                    aa = (av[0], av[1], av[2], av[3], av[4])
                    for gg in range(dout // 32):
                        acc_a = None
                        acc_b = None
                        for r in range(R):
                            w = rowsv[b][i, pl.ds(r * rw + gg * 16, 16)]
                            bf = plsc.bitcast(w, jnp.bfloat16)
                            pa, pb = plsc.unpack(
                                bf, format=plsc.PackFormat.INTERLEAVED)
                            if r == 0:
                                acc_a = aa[r] * pa
                                acc_b = aa[r] * pb
                            else:
                                acc_a = acc_a + aa[r] * pa
                                acc_b = acc_b + aa[r] * pb
                        msgv[b][i, pl.ds(gg * 32, 16)] = acc_a
                        msgv[b][i, pl.ds(gg * 32 + 16, 16)] = acc_b
                    return carry3

                lax.fori_loop(0, GB, edge, 0, unroll=2)

                for t in range(GB // 16):
                    dstr[b][pl.ds(t * 16, 16)] = (
                        dstsup[pl.ds(ch * GB + t * 16, 16)])
                pltpu.async_copy(msgv[b], agg.at[dstr[b]], sems[b],
                                 add=True)

                @pl.when(ch + 2 < CPS)
                def _():
                    pltpu.async_copy(
                        y.at[srcsup.at[pl.ds((ch + 2) * GB, GB)]],
                        rowsv[b], semg[b])
            return carry2

        lax.fori_loop(0, PAIRS, pair, 0)
        return carry

    lax.fori_loop(0, NSUP, sup_body, 0)

    for b in range(2):
        pltpu.make_async_copy(msgv[b], agg.at[dstr[0]], sems[b]).wait()

    plsc.subcore_barrier()

    @pl.when(s == 0)
    def _():
        pltpu.sync_copy(agg, out.at[c])


def _make_edge(dout):
    dyw = R * dout // 2
    return pl.kernel(
        functools.partial(_edge_body, dout),
        out_type=jax.ShapeDtypeStruct((NC, N, dout), _F32),
        mesh=_mesh(),
        compiler_params=_SC_PARAMS,
        scratch_types=[
            pltpu.VMEM((SUPER,), jnp.int32),
            pltpu.VMEM((SUPER,), jnp.int32),
            pltpu.VMEM((SUPER * R + 16,), _F32),
            pltpu.VMEM((GB, dyw), jnp.int32),
            pltpu.VMEM((GB, dyw), jnp.int32),
            pltpu.VMEM((GB, dout), _F32),
            pltpu.VMEM((GB, dout), _F32),
            pltpu.VMEM((GB,), jnp.int32),
            pltpu.VMEM((GB,), jnp.int32),
            pltpu.VMEM_SHARED((N, dout), _F32),
            pltpu.SemaphoreType.DMA,
            pltpu.SemaphoreType.DMA,
            pltpu.SemaphoreType.DMA,
            pltpu.SemaphoreType.DMA,
        ],
    )


def _pool_body(h, bat, zs, zc, outs, outc, idxv, hv, onev, sums, cnts, sem):
    del sem
    c = lax.axis_index("c")
    s = lax.axis_index("s")
    wid = s * NC + c

    @pl.when(s == 0)
    def _():
        pltpu.sync_copy(zs, sums)
        pltpu.sync_copy(zc, cnts)

    def fill(i, carry):
        onev[i, pl.ds(0, 16)] = jnp.full((16,), 1.0, _F32)
        return carry

    lax.fori_loop(0, NPW, fill, 0)
    plsc.subcore_barrier()

    base = wid * NPW
    pltpu.sync_copy(bat.at[pl.ds(base, NPW)], idxv)
    pltpu.sync_copy(h.at[pl.ds(base, NPW)], hv)
    pltpu.sync_copy(hv, sums.at[idxv], add=True)
    pltpu.sync_copy(onev, cnts.at[idxv], add=True)
    plsc.subcore_barrier()

    @pl.when(s == 0)
    def _():
        pltpu.sync_copy(sums, outs.at[c])
        pltpu.sync_copy(cnts, outc.at[c])


def _make_pool():
    return pl.kernel(
        _pool_body,
        out_type=(jax.ShapeDtypeStruct((NC, SEGPAD, 64), _F32),
                  jax.ShapeDtypeStruct((NC, SEGPAD, 16), _F32)),
        mesh=_mesh(),
        compiler_params=_SC_PARAMS,
        scratch_types=[
            pltpu.VMEM((NPW,), jnp.int32),
            pltpu.VMEM((NPW, 64), _F32),
            pltpu.VMEM((NPW, 16), _F32),
            pltpu.VMEM_SHARED((SEGPAD, 64), _F32),
            pltpu.VMEM_SHARED((SEGPAD, 16), _F32),
            pltpu.SemaphoreType.DMA,
        ],
    )


_PREC = lax.Precision.HIGHEST


def _elu(v):
    return jnp.where(v > 0, v, jnp.exp(jnp.minimum(v, 0.0)) - 1.0)


def _t0_body(emb_ref, ws_ref, wr_ref, b_ref, ty_ref, tr_ref):
    vals = lax.broadcasted_iota(jnp.int32, (100, 1), 0).astype(_F32)
    base = jnp.concatenate([emb_ref[...], vals], axis=1)
    ty_ref[...] = jnp.dot(base, ws_ref[...], precision=_PREC,
                          preferred_element_type=_F32)
    tr_ref[...] = (jnp.dot(base, wr_ref[...], precision=_PREC,
                           preferred_element_type=_F32) + b_ref[...])


def _t0_call(emb, ws0, wr0, b0):
    dy, dr = ws0.shape[1], wr0.shape[1]
    return pl.pallas_call(
        _t0_body,
        out_shape=(jax.ShapeDtypeStruct((100, dy), _F32),
                   jax.ShapeDtypeStruct((100, dr), _F32)),
    )(emb, ws0, wr0, b0)


_NODE_BLK = 1000


def _node_body(agg_ref, root_ref, ws_ref, wr_ref, b_ref, y_ref, rt_ref):
    h = _elu(agg_ref[0] + agg_ref[1] + root_ref[...])
    y_ref[...] = jnp.dot(h, ws_ref[...], precision=_PREC,
                         preferred_element_type=_F32).astype(jnp.bfloat16)
    rt_ref[...] = (jnp.dot(h, wr_ref[...], precision=_PREC,
                           preferred_element_type=_F32) + b_ref[...])


def _node_call(agg, root, ws, wr, b):
    dp = root.shape[1]
    dy, dr = ws.shape[1], wr.shape[1]
    nblk = N // _NODE_BLK
    return pl.pallas_call(
        _node_body,
        grid=(nblk,),
        in_specs=[
            pl.BlockSpec((NC, _NODE_BLK, dp), lambda i: (0, i, 0)),
            pl.BlockSpec((_NODE_BLK, dp), lambda i: (i, 0)),
            pl.BlockSpec((dp, dy), lambda i: (0, 0)),
            pl.BlockSpec((dp, dr), lambda i: (0, 0)),
            pl.BlockSpec((dr,), lambda i: (0,)),
        ],
        out_specs=(
            pl.BlockSpec((_NODE_BLK, dy), lambda i: (i, 0)),
            pl.BlockSpec((_NODE_BLK, dr), lambda i: (i, 0)),
        ),
        out_shape=(jax.ShapeDtypeStruct((N, dy), jnp.bfloat16),
                   jax.ShapeDtypeStruct((N, dr), _F32)),
    )(agg, root, ws, wr, b)


def _node5_body(agg_ref, root_ref, h_ref):
    h_ref[...] = _elu(agg_ref[0] + agg_ref[1] + root_ref[...])


def _node5_call(agg, root):
    dp = root.shape[1]
    nblk = N // _NODE_BLK
    return pl.pallas_call(
        _node5_body,
        grid=(nblk,),
        in_specs=[
            pl.BlockSpec((NC, _NODE_BLK, dp), lambda i: (0, i, 0)),
            pl.BlockSpec((_NODE_BLK, dp), lambda i: (i, 0)),
        ],
        out_specs=pl.BlockSpec((_NODE_BLK, dp), lambda i: (i, 0)),
        out_shape=jax.ShapeDtypeStruct((N, dp), _F32),
    )(agg, root)


def _mlp_body(s_ref, c_ref, w1_ref, b1_ref, w2_ref, b2_ref, w3_ref, b3_ref,
              out_ref):
    sums = (s_ref[0] + s_ref[1])[:NUM_GRAPHS]
    cnt = (c_ref[0] + c_ref[1])[:NUM_GRAPHS, 0:1]
    g = sums / jnp.maximum(cnt, 1.0)
    g = _elu(jnp.dot(g, w1_ref[...], precision=_PREC,
                     preferred_element_type=_F32) + b1_ref[...])
    g = _elu(jnp.dot(g, w2_ref[...], precision=_PREC,
                     preferred_element_type=_F32) + b2_ref[...])
    out_ref[...] = (jnp.dot(g, w3_ref[...], precision=_PREC,
                            preferred_element_type=_F32) + b3_ref[...])


def _mlp_call(sums, cnts, w1, b1, w2, b2, w3, b3):
    return pl.pallas_call(
        _mlp_body,
        out_shape=jax.ShapeDtypeStruct((NUM_GRAPHS, 1), _F32),
    )(sums, cnts, w1, b1, w2, b2, w3, b3)


def kernel(x, edge_index, edge_attr, batch, emb,
           W_rel0, W_root0, b0, W_rel1, W_root1, b1, W_rel2, W_root2, b2,
           W_rel3, W_root3, b3, W_rel4, W_root4, b4,
           fc1_w, fc1_b, fc2_w, fc2_b, fc3_w, fc3_b):
    src = edge_index[0]
    dst = edge_index[1]
    attr_flat = edge_attr.reshape(E * R)

    def stack(w):
        r, din, dout = w.shape
        s = jnp.transpose(w, (1, 0, 2)).reshape(din, r * dout)
        idx = []
        for rr in range(r):
            for gg in range(dout // 32):
                base = rr * dout + gg * 32
                for t in range(16):
                    idx += [base + t, base + 16 + t]
        return s[:, jnp.array(idx, dtype=jnp.int32)]

    def pack_bf16(a):
        n, m = a.shape
        a = a.astype(jnp.bfloat16)
        return lax.bitcast_convert_type(a.reshape(n, m // 2, 2), jnp.int32)

    ws = [stack(w) for w in (W_rel0, W_rel1, W_rel2, W_rel3, W_rel4)]
    wr = [W_root0, W_root1, W_root2, W_root3, W_root4]
    bs = [b0, b1, b2, b3, b4]
    douts = [w.shape[1] for w in wr]

    t_y, t_r = _t0_call(emb, ws[0], wr[0], bs[0])
    xpad = jnp.pad(x, (0, NPAD - N))
    ypad, rpad = _make_gather0(R * douts[0] // 2, douts[0])(
        pack_bf16(t_y), t_r, xpad)
    root = rpad[:N]
    y32 = ypad

    zer = {d: jnp.zeros((N, d), _F32) for d in (32, 64)}
    agg = _make_edge(douts[0])(y32, src, dst, attr_flat, zer[douts[0]])

    for l in range(1, 5):
        y, root = _node_call(agg, root, ws[l], wr[l], bs[l])
        agg = _make_edge(douts[l])(pack_bf16(y), src, dst, attr_flat,
                                   zer[douts[l]])

    h5 = _node5_call(agg, root)

    h5pad = jnp.pad(h5, ((0, NPAD - N), (0, 0)))
    batpad = jnp.concatenate(
        [batch, NUM_GRAPHS + (jnp.arange(NPAD - N, dtype=jnp.int32) % 16)])
    zs = jnp.zeros((SEGPAD, 64), _F32)
    zc = jnp.zeros((SEGPAD, 16), _F32)
    sums, cnts = _make_pool()(h5pad, batpad, zs, zc)

    return _mlp_call(sums, cnts, fc1_w, fc1_b, fc2_w, fc2_b, fc3_w, fc3_b)
